# Initial kernel scaffold; baseline (speedup 1.0000x reference)
import jax
import jax.numpy as jnp
from jax.experimental import pallas as pl


def kernel(loss, idx, weights, eta_value):
    raise NotImplementedError("write your pallas kernel here")


# speedup vs baseline: 1.5576x; 1.5576x over previous
import functools

import jax
import jax.numpy as jnp
from jax import lax
from jax.experimental import pallas as pl
from jax.experimental.pallas import tpu as pltpu
from jax.experimental.pallas import tpu_sc as plsc

B = 16384
M = 1000000
NW = 32
BP = B // NW
NCH = BP // 128
L = 16

_mesh = plsc.VectorSubcoreMesh(core_axis_name="c", subcore_axis_name="s")


@functools.partial(
    pl.kernel,
    out_type=jax.ShapeDtypeStruct((B,), jnp.float32),
    mesh=_mesh,
    scratch_types=[
        pltpu.VMEM((BP,), jnp.float32),
        pltpu.VMEM((NCH, 128), jnp.int32),
        pltpu.VMEM((NCH, 128), jnp.float32),
        pltpu.VMEM((BP,), jnp.float32),
        pltpu.VMEM((L,), jnp.float32),
        pltpu.SemaphoreType.DMA,
        pltpu.SemaphoreType.DMA,
    ],
)
def _sc_body(loss_hbm, idx_hbm, eta_hbm, w_ref, out_hbm,
             loss_v, idx_v, w_v, o_v, eta_v, in_sem, out_sem):
    wid = lax.axis_index("c") * 16 + lax.axis_index("s")
    base = wid * BP

    c_loss = pltpu.async_copy(loss_hbm.at[pl.ds(base, BP)], loss_v, in_sem)
    c_idx = pltpu.async_copy(idx_hbm.at[pl.ds(wid * NCH, NCH), :], idx_v,
                             in_sem)
    c_eta = pltpu.async_copy(eta_hbm, eta_v, in_sem)
    c_loss.wait()
    c_idx.wait()
    c_eta.wait()

    eta = eta_v[...]
    for i in range(BP // L):
        lv = loss_v[pl.ds(i * L, L)]
        t = 1.0 - lv / eta
        s = 1.0 / (1.0 + jnp.exp(-t))
        w = jnp.where(lv > eta, 0.0, s)
        o = 1.0 / (1.0 + jnp.exp(-w))
        j, r = (i * L) // 128, (i * L) % 128
        w_v[j, pl.ds(r, L)] = w
        o_v[pl.ds(i * L, L)] = o

    copies = [pltpu.async_copy(o_v, out_hbm.at[pl.ds(base, BP)], out_sem)]
    copies += [pltpu.async_copy(w_v.at[j], w_ref.at[idx_v.at[j]], out_sem)
               for j in range(NCH)]
    for c in copies:
        c.wait()


def kernel(loss, idx, weights, eta_value):
    eta16 = jnp.broadcast_to(eta_value, (L,))
    idx2d = idx.reshape(NW * NCH, 128)
    w_ref = jax.new_ref(weights)
    out = _sc_body(loss, idx2d, eta16, w_ref)
    return out, jax.freeze(w_ref)
